# baseline (speedup 1.0000x reference)
import jax
import jax.numpy as jnp
from jax import lax
from jax.experimental import pallas as pl
from jax.experimental.pallas import tpu as pltpu

_B, _H, _W = 4, 224, 224
_HP, _WP = 240, 256
_NEG = -1e30


def _roll(a, s, axis):
    return pltpu.roll(a, s % a.shape[axis], axis)


def _nms_kernel(x_ref, out_ref, v_ref, keep_ref):
    x = x_ref[...]

    xr = _roll(x, -1, 2)
    xl = _roll(x, 1, 2)
    xd = _roll(x, -1, 1)
    xu = _roll(x, 1, 1)
    rg_x = xr > x
    ll_x = x <= xl
    rg_y = xd > x
    ll_y = x <= xu
    neg = x <= 0
    valleys = rg_x & ll_x & rg_y & ll_y & neg
    peaks = (~rg_x) & (~ll_x) & (~rg_y) & (~ll_y) & (~neg)
    ext = peaks | valleys

    ri = lax.broadcasted_iota(jnp.int32, (_B, _HP, _WP), 1)
    ci = lax.broadcasted_iota(jnp.int32, (_B, _HP, _WP), 2)
    inb = (ri < _H) & (ci < _W)
    idx0 = ri * _WP + ci

    v_ref[...] = jnp.where(ext & inb, jnp.abs(x), _NEG)
    keep_ref[...] = jnp.zeros_like(x)

    def round_body(_):
        v = v_ref[...]
        mv, mi = v, idx0
        for axis in (1, 2):
            for s in (1, 2, 4):
                for sg in (s, -s):
                    bv = _roll(mv, sg, axis)
                    bi = _roll(mi, sg, axis)
                    tb = (bv > mv) | ((bv == mv) & (bi < mi))
                    mv = jnp.where(tb, bv, mv)
                    mi = jnp.where(tb, bi, mi)
        active = v >= 0
        k = active & (mv == v) & (mi == idx0)
        kf = k.astype(jnp.float32)
        keep_ref[...] = jnp.maximum(keep_ref[...], kf)
        d = kf
        for axis in (1, 2):
            for s in (1, 2, 4):
                for sg in (s, -s):
                    d = jnp.maximum(d, _roll(d, sg, axis))
        vn = jnp.where(d > 0, _NEG, v)
        v_ref[...] = vn
        return jnp.max(vn) >= 0

    lax.while_loop(lambda cont: cont, round_body,
                   jnp.max(v_ref[...]) >= 0)

    out_ref[...] = x * keep_ref[...]


def kernel(input_):
    x = input_.reshape(_B, _H, _W)
    inf = jnp.inf
    xp = jnp.zeros((_B, _HP, _WP), jnp.float32)
    xp = xp.at[:, :_H, :_W].set(x)
    xp = xp.at[:, :, _W].set(-inf)
    xp = xp.at[:, :, _WP - 1].set(inf)
    xp = xp.at[:, _H, :].set(-inf)
    xp = xp.at[:, _HP - 1, :].set(inf)

    out = pl.pallas_call(
        _nms_kernel,
        out_shape=jax.ShapeDtypeStruct((_B, _HP, _WP), jnp.float32),
        scratch_shapes=[
            pltpu.VMEM((_B, _HP, _WP), jnp.float32),
            pltpu.VMEM((_B, _HP, _WP), jnp.float32),
        ],
    )(xp)
    return out[:, :_H, :_W].reshape(input_.shape)


# speedup vs baseline: 1.3074x; 1.3074x over previous
import jax
import jax.numpy as jnp
from jax import lax
from jax.experimental import pallas as pl
from jax.experimental.pallas import tpu as pltpu

_B, _H, _W = 4, 224, 224
_NEG = -1e30
_BIGI = 1 << 30


def _shift(a, s, axis, fill):
    b, h, w = a.shape
    if axis == 1:
        pad_shape = (b, abs(s), w)
    else:
        pad_shape = (b, h, abs(s))
    pad = jnp.full(pad_shape, fill, a.dtype)
    if axis == 1:
        if s > 0:
            return jnp.concatenate([a[:, s:, :], pad], axis=1)
        return jnp.concatenate([pad, a[:, :s, :]], axis=1)
    else:
        if s > 0:
            return jnp.concatenate([a[:, :, s:], pad], axis=2)
        return jnp.concatenate([pad, a[:, :, :s]], axis=2)


def _nms_kernel(x_ref, out_ref, v_ref, keep_ref):
    x = x_ref[...]

    xr = jnp.concatenate([x[:, :, 1:], x[:, :, -1:]], axis=2)
    xl = jnp.concatenate([x[:, :, :1], x[:, :, :-1]], axis=2)
    xd = jnp.concatenate([x[:, 1:, :], x[:, -1:, :]], axis=1)
    xu = jnp.concatenate([x[:, :1, :], x[:, :-1, :]], axis=1)
    rg_x = xr > x
    ll_x = x <= xl
    rg_y = xd > x
    ll_y = x <= xu
    neg = x <= 0
    valleys = rg_x & ll_x & rg_y & ll_y & neg
    peaks = (~rg_x) & (~ll_x) & (~rg_y) & (~ll_y) & (~neg)
    ext = peaks | valleys

    v_ref[...] = jnp.where(ext, jnp.abs(x), _NEG)
    keep_ref[...] = jnp.zeros_like(x)

    ri = lax.broadcasted_iota(jnp.int32, (1, _H, _W), 1)
    ci = lax.broadcasted_iota(jnp.int32, (1, _H, _W), 2)
    idx0 = ri * _W + ci

    def round_body(_):
        v = v_ref[...]
        mv, mi = v, idx0
        for axis in (1, 2):
            for s in (1, 2, 4):
                for sg in (s, -s):
                    bv = _shift(mv, sg, axis, _NEG)
                    bi = _shift(mi, sg, axis, _BIGI)
                    tb = (bv > mv) | ((bv == mv) & (bi < mi))
                    mv = jnp.where(tb, bv, mv)
                    mi = jnp.where(tb, bi, mi)
        active = v >= 0
        k = active & (mv == v) & (mi == idx0)
        kf = k.astype(jnp.float32)
        keep_ref[...] = jnp.maximum(keep_ref[...], kf)
        d = kf
        for axis in (1, 2):
            for s in (1, 2, 4):
                for sg in (s, -s):
                    d = jnp.maximum(d, _shift(d, sg, axis, 0.0))
        vn = jnp.where(d > 0, _NEG, v)
        v_ref[...] = vn
        return jnp.max(vn) >= 0

    lax.while_loop(lambda cont: cont, round_body,
                   jnp.max(v_ref[...]) >= 0)

    out_ref[...] = x * keep_ref[...]


def kernel(input_):
    x = input_.reshape(_B, _H, _W)
    out = pl.pallas_call(
        _nms_kernel,
        grid=(_B,),
        in_specs=[pl.BlockSpec((1, _H, _W), lambda i: (i, 0, 0))],
        out_specs=pl.BlockSpec((1, _H, _W), lambda i: (i, 0, 0)),
        out_shape=jax.ShapeDtypeStruct((_B, _H, _W), jnp.float32),
        scratch_shapes=[
            pltpu.VMEM((1, _H, _W), jnp.float32),
            pltpu.VMEM((1, _H, _W), jnp.float32),
        ],
    )(x)
    return out.reshape(input_.shape)


# speedup vs baseline: 1.3358x; 1.0217x over previous
import jax
import jax.numpy as jnp
from jax import lax
from jax.experimental import pallas as pl
from jax.experimental.pallas import tpu as pltpu

_B, _H, _W = 4, 224, 224
_WB = _B * _W
_NEG = -1e30
_BIGI = 1 << 30


def _wcol(shape=( _H, _WB)):
    ci = lax.broadcasted_iota(jnp.int32, shape, 1)
    return ci % _W


def _shift_h(a, s, fill):
    h, w = a.shape
    pad = jnp.full((abs(s), w), fill, a.dtype)
    if s > 0:
        return jnp.concatenate([a[s:, :], pad], axis=0)
    return jnp.concatenate([pad, a[:s, :]], axis=0)


def _shift_w(a, s, fill, wc):
    h, w = a.shape
    pad = jnp.full((h, abs(s)), fill, a.dtype)
    if s > 0:
        moved = jnp.concatenate([a[:, s:], pad], axis=1)
        bad = wc >= (_W - s)
    else:
        moved = jnp.concatenate([pad, a[:, :s]], axis=1)
        bad = wc < (-s)
    return jnp.where(bad, jnp.full_like(a, fill), moved)


def _nms_kernel(x_ref, out_ref, v_ref, keep_ref):
    x = x_ref[...]
    wc = _wcol()

    inf = float("inf")
    xr = _shift_w(x, 1, -inf, wc)
    xl = _shift_w(x, -1, inf, wc)
    xd = _shift_h(x, 1, -inf)
    xu = _shift_h(x, -1, inf)
    rg_x = xr > x
    ll_x = x <= xl
    rg_y = xd > x
    ll_y = x <= xu
    neg = x <= 0
    valleys = rg_x & ll_x & rg_y & ll_y & neg
    peaks = (~rg_x) & (~ll_x) & (~rg_y) & (~ll_y) & (~neg)
    ext = peaks | valleys

    v_ref[...] = jnp.where(ext, jnp.abs(x), _NEG)
    keep_ref[...] = jnp.zeros_like(x)

    ri = lax.broadcasted_iota(jnp.int32, (_H, _WB), 0)
    ci = lax.broadcasted_iota(jnp.int32, (_H, _WB), 1)
    idx0 = ri * _WB + ci

    def round_body(_):
        v = v_ref[...]
        mv, mi = v, idx0
        for s in (1, 2, 4):
            for sg in (s, -s):
                bv = _shift_h(mv, sg, _NEG)
                bi = _shift_h(mi, sg, _BIGI)
                tb = (bv > mv) | ((bv == mv) & (bi < mi))
                mv = jnp.where(tb, bv, mv)
                mi = jnp.where(tb, bi, mi)
        for s in (1, 2, 4):
            for sg in (s, -s):
                bv = _shift_w(mv, sg, _NEG, wc)
                bi = _shift_w(mi, sg, _BIGI, wc)
                tb = (bv > mv) | ((bv == mv) & (bi < mi))
                mv = jnp.where(tb, bv, mv)
                mi = jnp.where(tb, bi, mi)
        active = v >= 0
        k = active & (mv == v) & (mi == idx0)
        kf = k.astype(jnp.float32)
        keep_ref[...] = jnp.maximum(keep_ref[...], kf)
        d = kf
        for s in (1, 2, 4):
            for sg in (s, -s):
                d = jnp.maximum(d, _shift_h(d, sg, 0.0))
        for s in (1, 2, 4):
            for sg in (s, -s):
                d = jnp.maximum(d, _shift_w(d, sg, 0.0, wc))
        vn = jnp.where(d > 0, _NEG, v)
        v_ref[...] = vn
        return jnp.max(vn) >= 0

    lax.while_loop(lambda cont: cont, round_body,
                   jnp.max(v_ref[...]) >= 0)

    out_ref[...] = x * keep_ref[...]


def kernel(input_):
    x = input_.reshape(_B, _H, _W).transpose(1, 0, 2).reshape(_H, _WB)
    out = pl.pallas_call(
        _nms_kernel,
        out_shape=jax.ShapeDtypeStruct((_H, _WB), jnp.float32),
        scratch_shapes=[
            pltpu.VMEM((_H, _WB), jnp.float32),
            pltpu.VMEM((_H, _WB), jnp.float32),
        ],
    )(x)
    return out.reshape(_H, _B, _W).transpose(1, 0, 2).reshape(input_.shape)


# speedup vs baseline: 1.5179x; 1.1364x over previous
import jax
import jax.numpy as jnp
from jax import lax
from jax.experimental import pallas as pl
from jax.experimental.pallas import tpu as pltpu

_B, _H, _W = 4, 224, 224
_NEG = -1e30
_BIGI = 1 << 30


def _shift(a, s, axis, fill):
    b, h, w = a.shape
    if axis == 1:
        pad_shape = (b, abs(s), w)
    else:
        pad_shape = (b, h, abs(s))
    pad = jnp.full(pad_shape, fill, a.dtype)
    if axis == 1:
        if s > 0:
            return jnp.concatenate([a[:, s:, :], pad], axis=1)
        return jnp.concatenate([pad, a[:, :s, :]], axis=1)
    else:
        if s > 0:
            return jnp.concatenate([a[:, :, s:], pad], axis=2)
        return jnp.concatenate([pad, a[:, :, :s]], axis=2)


def _nms_kernel(x_ref, out_ref, v_ref, keep_ref):
    x = x_ref[...]

    xr = jnp.concatenate([x[:, :, 1:], x[:, :, -1:]], axis=2)
    xl = jnp.concatenate([x[:, :, :1], x[:, :, :-1]], axis=2)
    xd = jnp.concatenate([x[:, 1:, :], x[:, -1:, :]], axis=1)
    xu = jnp.concatenate([x[:, :1, :], x[:, :-1, :]], axis=1)
    rg_x = xr > x
    ll_x = x <= xl
    rg_y = xd > x
    ll_y = x <= xu
    neg = x <= 0
    valleys = rg_x & ll_x & rg_y & ll_y & neg
    peaks = (~rg_x) & (~ll_x) & (~rg_y) & (~ll_y) & (~neg)
    ext = peaks | valleys

    v_ref[...] = jnp.where(ext, jnp.abs(x), _NEG)
    keep_ref[...] = jnp.zeros_like(x)

    ri = lax.broadcasted_iota(jnp.int32, (_B, _H, _W), 1)
    ci = lax.broadcasted_iota(jnp.int32, (_B, _H, _W), 2)
    idx0 = ri * _W + ci

    def round_body(_):
        v = v_ref[...]
        mv, mi = v, idx0
        for axis in (1, 2):
            for s in (1, 2, 4):
                for sg in (s, -s):
                    bv = _shift(mv, sg, axis, _NEG)
                    bi = _shift(mi, sg, axis, _BIGI)
                    tb = (bv > mv) | ((bv == mv) & (bi < mi))
                    mv = jnp.where(tb, bv, mv)
                    mi = jnp.where(tb, bi, mi)
        active = v >= 0
        k = active & (mv == v) & (mi == idx0)
        kf = k.astype(jnp.float32)
        keep_ref[...] = jnp.maximum(keep_ref[...], kf)
        d = kf
        for axis in (1, 2):
            for s in (1, 2, 4):
                for sg in (s, -s):
                    d = jnp.maximum(d, _shift(d, sg, axis, 0.0))
        vn = jnp.where(d > 0, _NEG, v)
        v_ref[...] = vn
        return jnp.max(vn) >= 0

    lax.while_loop(lambda cont: cont, round_body,
                   jnp.max(v_ref[...]) >= 0)

    out_ref[...] = x * keep_ref[...]


def kernel(input_):
    x = input_.reshape(_B, _H, _W)
    out = pl.pallas_call(
        _nms_kernel,
        out_shape=jax.ShapeDtypeStruct((_B, _H, _W), jnp.float32),
        scratch_shapes=[
            pltpu.VMEM((_B, _H, _W), jnp.float32),
            pltpu.VMEM((_B, _H, _W), jnp.float32),
        ],
    )(x)
    return out.reshape(input_.shape)
